# baseline (speedup 1.0000x reference)
import functools
import jax
import jax.numpy as jnp
from jax import lax
from jax.experimental import pallas as pl
from jax.experimental.pallas import tpu as pltpu, tpu_sc as plsc

_H = 16
_TOPK = 64
_TEMPERATURE = 1.0
_BQ = 256
_NC = 2
_NS = 16
_NW = _NC * _NS


def _qkv_body(x_ref, w_ref, b_ref, out_ref):
    x = x_ref[...]
    out_ref[0] = (
        jnp.dot(x, w_ref[0], preferred_element_type=jnp.float32) + b_ref[0][0][None, :]
    )


def _keys_body(q_ref, kt_ref, keys_ref, *, scale):
    s = jnp.dot(q_ref[0], kt_ref[0], preferred_element_type=jnp.float32) * scale
    si = jax.lax.bitcast_convert_type(s, jnp.int32)
    keys_ref[0] = si ^ jnp.where(
        si < 0, jnp.int32(-1), jnp.int32(-2147483648)
    )


def _sc_body(
    keys_hbm, thr_hbm, batch_v, cand_v, candb_v, hist2_v, hist_v, thr_v,
    *, rows, ss, topk, br
):
    rw = rows // _NW
    wid = lax.axis_index("s") * _NC + lax.axis_index("c")
    row0 = wid * rw
    iota = lax.iota(jnp.int32, 16)
    ones16 = jnp.ones((16,), jnp.int32)
    zeros16 = jnp.zeros((16,), jnp.int32)
    target = jnp.int32(ss - topk)

    def do_batch(b, _):
        base = row0 + b * br
        pltpu.sync_copy(keys_hbm.at[pl.ds(base, br)], batch_v)

        def z_step(g, _):
            hist2_v[pl.ds(g * 16, 16)] = zeros16
            return 0

        lax.fori_loop(0, 256, z_step, 0, unroll=8)

        def h_step(j, _):
            col = jnp.full((16,), j, jnp.int32)
            x = plsc.load_gather(batch_v, [iota, col])
            d = lax.shift_right_logical(x, 24)
            plsc.addupdate_scatter(hist2_v, [(d << 4) + iota], ones16)
            return 0

        lax.fori_loop(0, ss, h_step, 0, unroll=8)

        def f_step(bb, carry):
            acc, found, dstar, adstar = carry
            h = hist2_v[pl.ds(bb * 16, 16)]
            acc = acc + h
            hit = (acc > target) & jnp.logical_not(found)
            dstar = jnp.where(hit, bb, dstar)
            adstar = jnp.where(hit, acc, adstar)
            return acc, found | hit, dstar, adstar

        _, _, dstar_v, adstar_v = lax.fori_loop(
            0, 256, f_step,
            (zeros16, iota < 0, jnp.full((16,), 255, jnp.int32), zeros16),
            unroll=4,
        )
        r2_v = adstar_v - target

        def c_step(j, nout_v):
            col = jnp.full((16,), j, jnp.int32)
            x = plsc.load_gather(batch_v, [iota, col])
            d = lax.shift_right_logical(x, 24)
            msk = d == dstar_v
            plsc.store_scatter(cand_v, [iota, nout_v], x, mask=msk)
            return nout_v + jnp.where(msk, jnp.int32(1), jnp.int32(0))

        nout_v = lax.fori_loop(0, ss, c_step, zeros16, unroll=8)

        def do_row(rr, acc):
            sel = iota == rr
            nout_r = jnp.sum(jnp.where(sel, nout_v, 0))
            dstar_r = jnp.sum(jnp.where(sel, dstar_v, 0))
            r2_r = jnp.sum(jnp.where(sel, r2_v, 0))
            target2 = nout_r - r2_r

            for g in range(16):
                hist_v[pl.ds(g * 16, 16)] = zeros16

            nv1 = (nout_r + 15) >> 4

            def h2_step(j, _):
                x = cand_v[rr, pl.ds(j * 16, 16)]
                d2 = lax.shift_right_logical(x, 16) & jnp.int32(0xFF)
                valid = (iota + j * 16) < nout_r
                plsc.addupdate_scatter(hist_v, [d2], ones16, mask=valid)
                return 0

            lax.fori_loop(0, nv1, h2_step, 0)

            def cum_step(g, carry):
                v = hist_v[pl.ds(g * 16, 16)]
                hist_v[pl.ds(g * 16, 16)] = plsc.cumsum(v) + carry
                return carry + jnp.sum(v)

            lax.fori_loop(0, 16, cum_step, jnp.int32(0))

            def find_step(g, carry):
                dmin, amin = carry
                a = hist_v[pl.ds(g * 16, 16)]
                over = a > target2
                cand = jnp.where(over, iota + g * 16, jnp.int32(256))
                aval = jnp.where(over, a, jnp.int32(0x7FFFFFFF))
                return (
                    jnp.minimum(dmin, jnp.min(cand)),
                    jnp.minimum(amin, jnp.min(aval)),
                )

            d2star, a_d2star = lax.fori_loop(
                0, 16, find_step, (jnp.int32(256), jnp.int32(0x7FFFFFFF))
            )
            r3 = a_d2star - target2

            def c2_step(j, nb_v):
                x = cand_v[rr, pl.ds(j * 16, 16)]
                d2 = lax.shift_right_logical(x, 16) & jnp.int32(0xFF)
                valid = (iota + j * 16) < nout_r
                msk = (d2 == d2star) & valid
                mi = jnp.where(msk, jnp.int32(1), jnp.int32(0))
                pos = plsc.cumsum(mi) + (nb_v - 1)
                plsc.store_scatter(candb_v, [pos], x, mask=msk)
                return nb_v + plsc.all_reduce_population_count(msk)

            nb_v = lax.fori_loop(0, nv1, c2_step, zeros16)
            nv2 = (jnp.max(nb_v) + 15) >> 4
            m16 = jnp.int32(0xFFFF)

            def bis_step(_, lohi):
                lo, hi = lohi
                mid = (lo + hi) >> 1

                def cnt_step(j, c):
                    x = candb_v[pl.ds(j * 16, 16)]
                    valid = (iota + j * 16) < nb_v
                    hit = valid & ((x & m16) >= mid)
                    return c + plsc.all_reduce_population_count(hit)

                cnt = lax.fori_loop(0, nv2, cnt_step, zeros16)
                ge = cnt >= r3
                return jnp.where(ge, mid, lo), jnp.where(ge, hi, mid)

            lo, _ = lax.fori_loop(
                0, 16, bis_step,
                (zeros16, jnp.full((16,), 1 << 16, jnp.int32)),
            )
            thr = (dstar_r << 24) | (d2star << 16) | lo
            return jnp.where(sel, thr, acc)

        accf = lax.fori_loop(0, br, do_row, zeros16)
        thr_v[pl.ds(b * br, 16)] = accf
        return 0

    lax.fori_loop(0, rw // br, do_batch, 0)
    pltpu.sync_copy(thr_v, thr_hbm.at[pl.ds(row0, rw)])


def _attn_body(q_ref, kt_ref, v_ref, thr_ref, o_ref, *, topk, scale):
    s = jnp.dot(q_ref[0], kt_ref[0], preferred_element_type=jnp.float32) * scale
    ti = thr_ref[0][0] ^ jnp.int32(-2147483648)
    t = jax.lax.bitcast_convert_type(
        ti ^ ((ti >> 31) & jnp.int32(0x7FFFFFFF)), jnp.float32
    )[:, None]
    m = jnp.max(s, axis=1, keepdims=True)
    w = jnp.where(s >= t, jnp.exp(s - m), 0.0)
    denom = jnp.sum(w, axis=1, keepdims=True)
    attn = w * (1.0 / denom)
    o_ref[0] = jnp.dot(attn, v_ref[0], preferred_element_type=jnp.float32)


def _proj_body(x_ref, w_ref, b_ref, out_ref):
    out_ref[...] = (
        jnp.dot(x_ref[...], w_ref[...], preferred_element_type=jnp.float32)
        + b_ref[0][None, :]
    )


def kernel(x, Wq, bq, Wk, bk, Wv, bv, Wo, bo):
    b, s_len, d = x.shape
    h, dh = _H, d // _H
    scale = (dh ** -0.5) / _TEMPERATURE
    x2 = x.reshape(s_len, d)

    w3 = jnp.stack([Wq, Wk, Wv])
    b3 = jnp.stack([bq, bk, bv]).reshape(3, 1, d)

    nq = s_len // _BQ
    qkv = pl.pallas_call(
        _qkv_body,
        grid=(3, nq),
        in_specs=[
            pl.BlockSpec((_BQ, d), lambda j, i: (i, 0)),
            pl.BlockSpec((1, d, d), lambda j, i: (j, 0, 0)),
            pl.BlockSpec((1, 1, d), lambda j, i: (j, 0, 0)),
        ],
        out_specs=pl.BlockSpec((1, _BQ, d), lambda j, i: (j, i, 0)),
        out_shape=jax.ShapeDtypeStruct((3, s_len, d), jnp.float32),
    )(x2, w3, b3)

    q3 = qkv[0].reshape(s_len, h, dh).transpose(1, 0, 2)
    kt3 = qkv[1].reshape(s_len, h, dh).transpose(1, 2, 0)
    v3 = qkv[2].reshape(s_len, h, dh).transpose(1, 0, 2)

    keys = pl.pallas_call(
        functools.partial(_keys_body, scale=scale),
        grid=(h, nq),
        in_specs=[
            pl.BlockSpec((1, _BQ, dh), lambda hh, i: (hh, i, 0)),
            pl.BlockSpec((1, dh, s_len), lambda hh, i: (hh, 0, 0)),
        ],
        out_specs=pl.BlockSpec((1, _BQ, s_len), lambda hh, i: (hh, i, 0)),
        out_shape=jax.ShapeDtypeStruct((h, s_len, s_len), jnp.int32),
    )(q3, kt3)

    rows = h * s_len
    keys2 = keys.reshape(rows, s_len)
    br = 16
    mesh = plsc.VectorSubcoreMesh(core_axis_name="c", subcore_axis_name="s")
    thr = pl.kernel(
        functools.partial(_sc_body, rows=rows, ss=s_len, topk=_TOPK, br=br),
        out_type=jax.ShapeDtypeStruct((rows,), jnp.int32),
        mesh=mesh,
        compiler_params=pltpu.CompilerParams(needs_layout_passes=False),
        scratch_types=[
            pltpu.VMEM((br, s_len), jnp.int32),
            pltpu.VMEM((br, s_len), jnp.int32),
            pltpu.VMEM((s_len,), jnp.int32),
            pltpu.VMEM((4096,), jnp.int32),
            pltpu.VMEM((256,), jnp.int32),
            pltpu.VMEM((rows // _NW,), jnp.int32),
        ],
    )(keys2)

    thr4 = thr.reshape(h * nq, 1, _BQ)

    o3 = pl.pallas_call(
        functools.partial(_attn_body, topk=_TOPK, scale=scale),
        grid=(h, nq),
        in_specs=[
            pl.BlockSpec((1, _BQ, dh), lambda hh, i: (hh, i, 0)),
            pl.BlockSpec((1, dh, s_len), lambda hh, i: (hh, 0, 0)),
            pl.BlockSpec((1, s_len, dh), lambda hh, i: (hh, 0, 0)),
            pl.BlockSpec((1, 1, _BQ), lambda hh, i: (hh * (s_len // _BQ) + i, 0, 0)),
        ],
        out_specs=pl.BlockSpec((1, _BQ, dh), lambda hh, i: (hh, i, 0)),
        out_shape=jax.ShapeDtypeStruct((h, s_len, dh), jnp.float32),
    )(q3, kt3, v3, thr4)

    o2 = o3.transpose(1, 0, 2).reshape(s_len, d)

    out = pl.pallas_call(
        _proj_body,
        grid=(nq,),
        in_specs=[
            pl.BlockSpec((_BQ, d), lambda i: (i, 0)),
            pl.BlockSpec((d, d), lambda i: (0, 0)),
            pl.BlockSpec((1, d), lambda i: (0, 0)),
        ],
        out_specs=pl.BlockSpec((_BQ, d), lambda i: (i, 0)),
        out_shape=jax.ShapeDtypeStruct((s_len, d), jnp.float32),
    )(o2, Wo, bo.reshape(1, d))

    return out.reshape(b, s_len, d)


# speedup vs baseline: 3.4135x; 3.4135x over previous
import functools
import jax
import jax.numpy as jnp
from jax.experimental import pallas as pl

_H = 16
_TOPK = 64
_TEMPERATURE = 1.0
_BQ = 512


def _qkv_body(x_ref, w_ref, b_ref, out_ref):
    x = x_ref[...]
    w = w_ref[0]
    b = b_ref[0]
    out_ref[0] = jnp.dot(x, w, preferred_element_type=jnp.float32) + b[0][None, :]


def _attn_body(q_ref, kt_ref, v_ref, o_ref, *, topk, scale):
    q = q_ref[0]
    kt = kt_ref[0]
    v = v_ref[0]
    s = jnp.dot(q, kt, preferred_element_type=jnp.float32) * scale

    def to_f32(k):
        return jax.lax.bitcast_convert_type(
            k ^ ((k >> 31) & jnp.int32(0x7FFFFFFF)), jnp.float32
        )

    bq = s.shape[0]
    lo0 = jnp.full((bq, 1), jnp.int32(-2139095041), jnp.int32)
    hi0 = jnp.full((bq, 1), jnp.int32(0x7F800000), jnp.int32)

    def step(_, carry):
        lo, hi = carry
        mid = (lo >> 1) + (hi >> 1) + (lo & hi & 1)
        cnt = jnp.sum((s >= to_f32(mid)).astype(jnp.int32), axis=1, keepdims=True)
        ge = cnt >= topk
        return jnp.where(ge, mid, lo), jnp.where(ge, hi, mid)

    lo, hi = jax.lax.fori_loop(0, 32, step, (lo0, hi0))
    t = to_f32(lo)
    m = jnp.max(s, axis=1, keepdims=True)
    w = jnp.where(s >= t, jnp.exp(s - m), 0.0)
    denom = jnp.sum(w, axis=1, keepdims=True)
    attn = w / denom
    o_ref[0] = jnp.dot(attn, v, preferred_element_type=jnp.float32)


def _proj_body(x_ref, w_ref, b_ref, out_ref):
    out_ref[...] = (
        jnp.dot(x_ref[...], w_ref[...], preferred_element_type=jnp.float32)
        + b_ref[0][None, :]
    )


def kernel(x, Wq, bq, Wk, bk, Wv, bv, Wo, bo):
    b, s_len, d = x.shape
    h, dh = _H, d // _H
    scale = (dh ** -0.5) / _TEMPERATURE
    x2 = x.reshape(s_len, d)

    w3 = jnp.stack([Wq, Wk, Wv])
    b3 = jnp.stack([bq, bk, bv]).reshape(3, 1, d)

    nq = s_len // _BQ
    qkv = pl.pallas_call(
        _qkv_body,
        grid=(3, nq),
        in_specs=[
            pl.BlockSpec((_BQ, d), lambda j, i: (i, 0)),
            pl.BlockSpec((1, d, d), lambda j, i: (j, 0, 0)),
            pl.BlockSpec((1, 1, d), lambda j, i: (j, 0, 0)),
        ],
        out_specs=pl.BlockSpec((1, _BQ, d), lambda j, i: (j, i, 0)),
        out_shape=jax.ShapeDtypeStruct((3, s_len, d), jnp.float32),
    )(x2, w3, b3)

    q3 = qkv[0].reshape(s_len, h, dh).transpose(1, 0, 2)
    kt3 = qkv[1].reshape(s_len, h, dh).transpose(1, 2, 0)
    v3 = qkv[2].reshape(s_len, h, dh).transpose(1, 0, 2)

    o3 = pl.pallas_call(
        functools.partial(_attn_body, topk=_TOPK, scale=scale),
        grid=(h, nq),
        in_specs=[
            pl.BlockSpec((1, _BQ, dh), lambda hh, i: (hh, i, 0)),
            pl.BlockSpec((1, dh, s_len), lambda hh, i: (hh, 0, 0)),
            pl.BlockSpec((1, s_len, dh), lambda hh, i: (hh, 0, 0)),
        ],
        out_specs=pl.BlockSpec((1, _BQ, dh), lambda hh, i: (hh, i, 0)),
        out_shape=jax.ShapeDtypeStruct((h, s_len, dh), jnp.float32),
    )(q3, kt3, v3)

    o2 = o3.transpose(1, 0, 2).reshape(s_len, d)

    out = pl.pallas_call(
        _proj_body,
        grid=(nq,),
        in_specs=[
            pl.BlockSpec((_BQ, d), lambda i: (i, 0)),
            pl.BlockSpec((d, d), lambda i: (0, 0)),
            pl.BlockSpec((1, d), lambda i: (0, 0)),
        ],
        out_specs=pl.BlockSpec((_BQ, d), lambda i: (i, 0)),
        out_shape=jax.ShapeDtypeStruct((s_len, d), jnp.float32),
    )(o2, Wo, bo.reshape(1, d))

    return out.reshape(b, s_len, d)


# speedup vs baseline: 3.5449x; 1.0385x over previous
import functools
import jax
import jax.numpy as jnp
from jax.experimental import pallas as pl

_H = 16
_TOPK = 64
_TEMPERATURE = 1.0
_BQ = 1024


def _qkv_body(x_ref, w_ref, b_ref, out_ref):
    x = x_ref[...]
    w = w_ref[0]
    b = b_ref[0]
    out_ref[0] = jnp.dot(x, w, preferred_element_type=jnp.float32) + b[0][None, :]


def _attn_body(q_ref, kt_ref, v_ref, o_ref, *, topk, scale):
    q = q_ref[0]
    kt = kt_ref[0]
    v = v_ref[0]
    s = jnp.dot(q, kt, preferred_element_type=jnp.float32) * scale

    def to_f32(k):
        return jax.lax.bitcast_convert_type(
            k ^ ((k >> 31) & jnp.int32(0x7FFFFFFF)), jnp.float32
        )

    bq = s.shape[0]
    lo0 = jnp.full((bq, 1), jnp.int32(-2139095041), jnp.int32)
    hi0 = jnp.full((bq, 1), jnp.int32(0x7F800000), jnp.int32)

    def step(_, carry):
        lo, hi = carry
        mid = (lo >> 1) + (hi >> 1) + (lo & hi & 1)
        cnt = jnp.sum((s >= to_f32(mid)).astype(jnp.int32), axis=1, keepdims=True)
        ge = cnt >= topk
        return jnp.where(ge, mid, lo), jnp.where(ge, hi, mid)

    lo, hi = jax.lax.fori_loop(0, 32, step, (lo0, hi0))
    t = to_f32(lo)
    m = jnp.max(s, axis=1, keepdims=True)
    w = jnp.where(s >= t, jnp.exp(s - m), 0.0)
    denom = jnp.sum(w, axis=1, keepdims=True)
    attn = w / denom
    o_ref[0] = jnp.dot(attn, v, preferred_element_type=jnp.float32)


def _proj_body(x_ref, w_ref, b_ref, out_ref):
    out_ref[...] = (
        jnp.dot(x_ref[...], w_ref[...], preferred_element_type=jnp.float32)
        + b_ref[0][None, :]
    )


def kernel(x, Wq, bq, Wk, bk, Wv, bv, Wo, bo):
    b, s_len, d = x.shape
    h, dh = _H, d // _H
    scale = (dh ** -0.5) / _TEMPERATURE
    x2 = x.reshape(s_len, d)

    w3 = jnp.stack([Wq, Wk, Wv])
    b3 = jnp.stack([bq, bk, bv]).reshape(3, 1, d)

    nq = s_len // _BQ
    qkv = pl.pallas_call(
        _qkv_body,
        grid=(3, nq),
        in_specs=[
            pl.BlockSpec((_BQ, d), lambda j, i: (i, 0)),
            pl.BlockSpec((1, d, d), lambda j, i: (j, 0, 0)),
            pl.BlockSpec((1, 1, d), lambda j, i: (j, 0, 0)),
        ],
        out_specs=pl.BlockSpec((1, _BQ, d), lambda j, i: (j, i, 0)),
        out_shape=jax.ShapeDtypeStruct((3, s_len, d), jnp.float32),
    )(x2, w3, b3)

    q3 = qkv[0].reshape(s_len, h, dh).transpose(1, 0, 2)
    kt3 = qkv[1].reshape(s_len, h, dh).transpose(1, 2, 0)
    v3 = qkv[2].reshape(s_len, h, dh).transpose(1, 0, 2)

    o3 = pl.pallas_call(
        functools.partial(_attn_body, topk=_TOPK, scale=scale),
        grid=(h, nq),
        in_specs=[
            pl.BlockSpec((1, _BQ, dh), lambda hh, i: (hh, i, 0)),
            pl.BlockSpec((1, dh, s_len), lambda hh, i: (hh, 0, 0)),
            pl.BlockSpec((1, s_len, dh), lambda hh, i: (hh, 0, 0)),
        ],
        out_specs=pl.BlockSpec((1, _BQ, dh), lambda hh, i: (hh, i, 0)),
        out_shape=jax.ShapeDtypeStruct((h, s_len, dh), jnp.float32),
    )(q3, kt3, v3)

    o2 = o3.transpose(1, 0, 2).reshape(s_len, d)

    out = pl.pallas_call(
        _proj_body,
        grid=(nq,),
        in_specs=[
            pl.BlockSpec((_BQ, d), lambda i: (i, 0)),
            pl.BlockSpec((d, d), lambda i: (0, 0)),
            pl.BlockSpec((1, d), lambda i: (0, 0)),
        ],
        out_specs=pl.BlockSpec((_BQ, d), lambda i: (i, 0)),
        out_shape=jax.ShapeDtypeStruct((s_len, d), jnp.float32),
    )(o2, Wo, bo.reshape(1, d))

    return out.reshape(b, s_len, d)


# speedup vs baseline: 3.6069x; 1.0175x over previous
import functools
import jax
import jax.numpy as jnp
from jax.experimental import pallas as pl

_H = 16
_TOPK = 64
_TEMPERATURE = 1.0
_BQ = 2048


def _qkv_body(x_ref, w_ref, b_ref, out_ref):
    x = x_ref[...]
    w = w_ref[0]
    b = b_ref[0]
    out_ref[0] = jnp.dot(x, w, preferred_element_type=jnp.float32) + b[0][None, :]


def _attn_body(q_ref, kt_ref, v_ref, o_ref, *, topk, scale):
    q = q_ref[0]
    kt = kt_ref[0]
    v = v_ref[0]
    s = jnp.dot(q, kt, preferred_element_type=jnp.float32) * scale

    def to_f32(k):
        return jax.lax.bitcast_convert_type(
            k ^ ((k >> 31) & jnp.int32(0x7FFFFFFF)), jnp.float32
        )

    bq = s.shape[0]
    lo0 = jnp.full((bq, 1), jnp.int32(-2139095041), jnp.int32)
    hi0 = jnp.full((bq, 1), jnp.int32(0x7F800000), jnp.int32)

    def step(_, carry):
        lo, hi = carry
        mid = (lo >> 1) + (hi >> 1) + (lo & hi & 1)
        cnt = jnp.sum((s >= to_f32(mid)).astype(jnp.int32), axis=1, keepdims=True)
        ge = cnt >= topk
        return jnp.where(ge, mid, lo), jnp.where(ge, hi, mid)

    lo, hi = jax.lax.fori_loop(0, 32, step, (lo0, hi0))
    t = to_f32(lo)
    m = jnp.max(s, axis=1, keepdims=True)
    w = jnp.where(s >= t, jnp.exp(s - m), 0.0)
    denom = jnp.sum(w, axis=1, keepdims=True)
    attn = w / denom
    o_ref[0] = jnp.dot(attn, v, preferred_element_type=jnp.float32)


def _proj_body(x_ref, w_ref, b_ref, out_ref):
    out_ref[...] = (
        jnp.dot(x_ref[...], w_ref[...], preferred_element_type=jnp.float32)
        + b_ref[0][None, :]
    )


def kernel(x, Wq, bq, Wk, bk, Wv, bv, Wo, bo):
    b, s_len, d = x.shape
    h, dh = _H, d // _H
    scale = (dh ** -0.5) / _TEMPERATURE
    x2 = x.reshape(s_len, d)

    w3 = jnp.stack([Wq, Wk, Wv])
    b3 = jnp.stack([bq, bk, bv]).reshape(3, 1, d)

    nq = s_len // _BQ
    qkv = pl.pallas_call(
        _qkv_body,
        grid=(3, nq),
        in_specs=[
            pl.BlockSpec((_BQ, d), lambda j, i: (i, 0)),
            pl.BlockSpec((1, d, d), lambda j, i: (j, 0, 0)),
            pl.BlockSpec((1, 1, d), lambda j, i: (j, 0, 0)),
        ],
        out_specs=pl.BlockSpec((1, _BQ, d), lambda j, i: (j, i, 0)),
        out_shape=jax.ShapeDtypeStruct((3, s_len, d), jnp.float32),
    )(x2, w3, b3)

    q3 = qkv[0].reshape(s_len, h, dh).transpose(1, 0, 2)
    kt3 = qkv[1].reshape(s_len, h, dh).transpose(1, 2, 0)
    v3 = qkv[2].reshape(s_len, h, dh).transpose(1, 0, 2)

    o3 = pl.pallas_call(
        functools.partial(_attn_body, topk=_TOPK, scale=scale),
        grid=(h, nq),
        in_specs=[
            pl.BlockSpec((1, _BQ, dh), lambda hh, i: (hh, i, 0)),
            pl.BlockSpec((1, dh, s_len), lambda hh, i: (hh, 0, 0)),
            pl.BlockSpec((1, s_len, dh), lambda hh, i: (hh, 0, 0)),
        ],
        out_specs=pl.BlockSpec((1, _BQ, dh), lambda hh, i: (hh, i, 0)),
        out_shape=jax.ShapeDtypeStruct((h, s_len, dh), jnp.float32),
    )(q3, kt3, v3)

    o2 = o3.transpose(1, 0, 2).reshape(s_len, d)

    out = pl.pallas_call(
        _proj_body,
        grid=(nq,),
        in_specs=[
            pl.BlockSpec((_BQ, d), lambda i: (i, 0)),
            pl.BlockSpec((d, d), lambda i: (0, 0)),
            pl.BlockSpec((1, d), lambda i: (0, 0)),
        ],
        out_specs=pl.BlockSpec((_BQ, d), lambda i: (i, 0)),
        out_shape=jax.ShapeDtypeStruct((s_len, d), jnp.float32),
    )(o2, Wo, bo.reshape(1, d))

    return out.reshape(b, s_len, d)
